# Initial kernel scaffold; baseline (speedup 1.0000x reference)
import jax
import jax.numpy as jnp
from jax.experimental import pallas as pl


def kernel(x, edge_index, edge_weight, W, b):
    raise NotImplementedError("write your pallas kernel here")


# speedup vs baseline: 2.9903x; 2.9903x over previous
import functools

import jax
import jax.numpy as jnp
from jax import lax
from jax.experimental import pallas as pl
from jax.experimental.pallas import tpu as pltpu
from jax.experimental.pallas import tpu_sc as plsc

NC = 2
NS = 16
L = 16


def _sc_aggregate(xcols, src, dst, w, n_nodes):
    N = n_nodes
    DH = xcols.shape[1]
    E = src.shape[0]
    EPW = E // NS
    K = 80
    NCHUNK = EPW // K
    assert EPW % K == 0 and E % NS == 0
    RPT = (N // NS) // 8 * 8
    REM = N - RPT * NS
    SEG = DH // L

    mesh = plsc.VectorSubcoreMesh(core_axis_name="c", subcore_axis_name="s")

    @functools.partial(
        pl.kernel,
        mesh=mesh,
        compiler_params=pltpu.CompilerParams(use_tc_tiling_on_sc=False),
        out_type=[
            jax.ShapeDtypeStruct((NC, N, DH), jnp.float32),
            jax.ShapeDtypeStruct((N, L), jnp.float32),
        ],
        scratch_types=[
            pltpu.VMEM((K,), jnp.int32),
            pltpu.VMEM((K,), jnp.int32),
            pltpu.VMEM((K,), jnp.float32),
            pltpu.VMEM((K, 64), jnp.float32),
            pltpu.VMEM((K, L), jnp.float32),
            pltpu.VMEM((624, L), jnp.float32),
            pltpu.VMEM_SHARED((N, 64), jnp.float32),
            pltpu.VMEM_SHARED((N, L), jnp.float32),
            pltpu.SemaphoreType.DMA,
        ],
    )
    def k(x_hbm, src_hbm, dst_hbm, w_hbm, acc_out, cnt_out,
          dstv, srcv, wv, rows, ones, zbuf, acc_sh, cnt_sh, sem):
        c = lax.axis_index("c")
        s = lax.axis_index("s")

        zero = jnp.zeros((L,), jnp.float32)
        onehot = jnp.where(jnp.arange(L, dtype=jnp.int32) == 0, 1.0, 0.0)

        def init_rows(i, _):
            for j in range(SEG):
                rows[i, pl.ds(j * L, L)] = zero
            ones[i, :] = onehot
            return 0
        lax.fori_loop(0, K, init_rows, 0)

        def init_z(i, _):
            zbuf[i, :] = zero
            return 0
        lax.fori_loop(0, RPT, init_z, 0)

        r0 = pl.multiple_of(s * RPT, 8)
        done = 0
        while done < RPT:
            step = min(K, RPT - done)
            pltpu.sync_copy(rows.at[pl.ds(0, step)],
                            acc_sh.at[pl.ds(pl.multiple_of(r0 + done, 8), step)])
            done += step

        @pl.when(c == 0)
        def _():
            pltpu.sync_copy(zbuf.at[pl.ds(0, RPT)], cnt_sh.at[pl.ds(r0, RPT)])

        @pl.when(s == NS - 1)
        def _():
            pltpu.sync_copy(rows.at[pl.ds(0, REM)],
                            acc_sh.at[pl.ds(RPT * NS, REM)])

            @pl.when(c == 0)
            def _():
                pltpu.sync_copy(zbuf.at[pl.ds(0, REM)],
                                cnt_sh.at[pl.ds(RPT * NS, REM)])

        plsc.subcore_barrier()

        ebase = s * EPW
        goff = c * N

        def chunk(i, _):
            base = ebase + i * K
            pltpu.sync_copy(dst_hbm.at[pl.ds(base, K)], dstv)
            pltpu.sync_copy(w_hbm.at[pl.ds(base, K)], wv)
            for g in range(K // L):
                dstv[pl.ds(g * L, L)] = dstv[pl.ds(g * L, L)] + goff
            pltpu.async_copy(x_hbm.at[dstv], rows, sem).wait()

            def scale(g, _):
                e0 = g * L
                w16 = wv[pl.ds(e0, L)]
                for i2 in range(L):
                    we = w16[i2]
                    for j in range(SEG):
                        seg = rows[e0 + i2, pl.ds(j * L, L)]
                        rows[e0 + i2, pl.ds(j * L, L)] = seg * we
                return 0
            lax.fori_loop(0, K // L, scale, 0)

            pltpu.sync_copy(src_hbm.at[pl.ds(base, K)], srcv)
            pltpu.sync_copy(rows, acc_sh.at[srcv], add=True)

            @pl.when(c == 0)
            def _():
                pltpu.sync_copy(ones, cnt_sh.at[srcv], add=True)
            return 0
        lax.fori_loop(0, NCHUNK, chunk, 0)

        plsc.subcore_barrier()
        pltpu.sync_copy(acc_sh.at[pl.ds(r0, RPT)], acc_out.at[c, pl.ds(r0, RPT)])

        @pl.when(c == 0)
        def _():
            pltpu.sync_copy(cnt_sh.at[pl.ds(r0, RPT)], cnt_out.at[pl.ds(r0, RPT)])

        @pl.when(s == NS - 1)
        def _():
            pltpu.sync_copy(acc_sh.at[pl.ds(RPT * NS, REM)],
                            acc_out.at[c, pl.ds(RPT * NS, REM)])

            @pl.when(c == 0)
            def _():
                pltpu.sync_copy(cnt_sh.at[pl.ds(RPT * NS, REM)],
                                cnt_out.at[pl.ds(RPT * NS, REM)])

    return k(xcols, src, dst, w)


def _tc_combine(x, acc, cnt, w1t, w2t, b2):
    N, D = x.shape
    OUT = w1t.shape[1]
    DH = D // 2
    BN = 400

    def body(x_ref, acc_ref, cnt_ref, w1_ref, w2_ref, b_ref, o_ref):
        xb = x_ref[...]
        a = jnp.concatenate([acc_ref[0], acc_ref[1]], axis=1)
        cn = cnt_ref[:, 0:1]
        agg = a * (1.0 / jnp.maximum(cn, 1.0))
        o_ref[...] = (
            jnp.dot(xb, w1_ref[...], preferred_element_type=jnp.float32)
            + jnp.dot(agg, w2_ref[...], preferred_element_type=jnp.float32)
            + b_ref[...]
        )

    return pl.pallas_call(
        body,
        grid=(N // BN,),
        in_specs=[
            pl.BlockSpec((BN, D), lambda i: (i, 0)),
            pl.BlockSpec((NC, BN, DH), lambda i: (0, i, 0)),
            pl.BlockSpec((BN, L), lambda i: (i, 0)),
            pl.BlockSpec((D, OUT), lambda i: (0, 0)),
            pl.BlockSpec((D, OUT), lambda i: (0, 0)),
            pl.BlockSpec((1, OUT), lambda i: (0, 0)),
        ],
        out_specs=pl.BlockSpec((BN, OUT), lambda i: (i, 0)),
        out_shape=jax.ShapeDtypeStruct((N, OUT), jnp.float32),
    )(x, acc, cnt, w1t, w2t, b2)


def kernel(x, edge_index, edge_weight, W, b):
    N, D = x.shape
    DH = D // 2
    src = edge_index[0]
    dst = edge_index[1]
    xcols = jnp.concatenate([x[:, :DH], x[:, DH:]], axis=0)
    acc, cnt = _sc_aggregate(xcols, src, dst, edge_weight, N)
    w1t = W[:, :D].T
    w2t = W[:, D:].T
    return _tc_combine(x, acc, cnt, w1t, w2t, b[None, :])


# speedup vs baseline: 8.2468x; 2.7578x over previous
import functools

import jax
import jax.numpy as jnp
from jax import lax
from jax.experimental import pallas as pl
from jax.experimental.pallas import tpu as pltpu
from jax.experimental.pallas import tpu_sc as plsc

NC = 2
NS = 16
L = 16


def _sc_aggregate(xcols, src, dst, w, n_nodes):
    N = n_nodes
    DH = xcols.shape[1]
    E = src.shape[0]
    EPW = E // NS
    K = 80
    NBUF = 4
    NCHUNK = EPW // K
    assert EPW % K == 0 and E % NS == 0
    RPT = (N // NS) // 8 * 8
    REM = N - RPT * NS
    SEG = DH // L

    mesh = plsc.VectorSubcoreMesh(core_axis_name="c", subcore_axis_name="s")

    @functools.partial(
        pl.kernel,
        mesh=mesh,
        compiler_params=pltpu.CompilerParams(use_tc_tiling_on_sc=False),
        out_type=[
            jax.ShapeDtypeStruct((NC, N, DH), jnp.float32),
            jax.ShapeDtypeStruct((N, L), jnp.float32),
        ],
        scratch_types=[
            pltpu.VMEM((NBUF, K), jnp.int32),
            pltpu.VMEM((NBUF, K), jnp.int32),
            pltpu.VMEM((NBUF, K), jnp.float32),
            pltpu.VMEM((NBUF, K, 64), jnp.float32),
            pltpu.VMEM((K, L), jnp.float32),
            pltpu.VMEM((624, L), jnp.float32),
            pltpu.VMEM_SHARED((N, 64), jnp.float32),
            pltpu.VMEM_SHARED((N, L), jnp.float32),
            pltpu.SemaphoreType.DMA((NBUF,)),
            pltpu.SemaphoreType.DMA((NBUF,)),
            pltpu.SemaphoreType.DMA((NBUF,)),
            pltpu.SemaphoreType.DMA((NBUF,)),
        ],
    )
    def k(x_hbm, src_hbm, dst_hbm, w_hbm, acc_out, cnt_out,
          dstv, srcv, wv, rows, ones, zbuf, acc_sh, cnt_sh,
          semi, semg, sems, semc):
        c = lax.axis_index("c")
        s = lax.axis_index("s")

        zero = jnp.zeros((L,), jnp.float32)
        onehot = jnp.where(jnp.arange(L, dtype=jnp.int32) == 0, 1.0, 0.0)

        def init_rows(i, _):
            for j in range(SEG):
                rows[0, i, pl.ds(j * L, L)] = zero
            ones[i, :] = onehot
            return 0
        lax.fori_loop(0, K, init_rows, 0)

        def init_z(i, _):
            zbuf[i, :] = zero
            return 0
        lax.fori_loop(0, RPT, init_z, 0)

        r0 = pl.multiple_of(s * RPT, 8)
        done = 0
        while done < RPT:
            step = min(K, RPT - done)
            pltpu.sync_copy(rows.at[0, pl.ds(0, step)],
                            acc_sh.at[pl.ds(pl.multiple_of(r0 + done, 8), step)])
            done += step

        @pl.when(c == 0)
        def _():
            pltpu.sync_copy(zbuf.at[pl.ds(0, RPT)], cnt_sh.at[pl.ds(r0, RPT)])

        @pl.when(s == NS - 1)
        def _():
            pltpu.sync_copy(rows.at[0, pl.ds(0, REM)],
                            acc_sh.at[pl.ds(RPT * NS, REM)])

            @pl.when(c == 0)
            def _():
                pltpu.sync_copy(zbuf.at[pl.ds(0, REM)],
                                cnt_sh.at[pl.ds(RPT * NS, REM)])

        plsc.subcore_barrier()

        ebase = s * EPW
        goff = c * N

        def pipe(g, _):
            for b in range(NBUF):
                i = g * NBUF + b
                bj = (b + 2) % NBUF
                bk = (b + 1) % NBUF

                @pl.when(jnp.logical_and(i >= NBUF, i < NCHUNK + NBUF))
                def _():
                    pltpu.make_async_copy(
                        rows.at[b], acc_sh.at[srcv.at[b]], sems.at[b]).wait()

                    @pl.when(c == 0)
                    def _():
                        pltpu.make_async_copy(
                            ones, cnt_sh.at[srcv.at[b]], semc.at[b]).wait()

                @pl.when(i < NCHUNK)
                def _():
                    base = ebase + i * K
                    pltpu.async_copy(dst_hbm.at[pl.ds(base, K)], dstv.at[b],
                                     semi.at[b])
                    pltpu.async_copy(w_hbm.at[pl.ds(base, K)], wv.at[b],
                                     semi.at[b])
                    pltpu.async_copy(src_hbm.at[pl.ds(base, K)], srcv.at[b],
                                     semi.at[b])

                @pl.when(jnp.logical_and(i >= 2, i < NCHUNK + 2))
                def _():
                    base = ebase + (i - 2) * K
                    pltpu.make_async_copy(dst_hbm.at[pl.ds(base, K)],
                                          dstv.at[bj], semi.at[bj]).wait()
                    pltpu.make_async_copy(w_hbm.at[pl.ds(base, K)],
                                          wv.at[bj], semi.at[bj]).wait()
                    pltpu.make_async_copy(src_hbm.at[pl.ds(base, K)],
                                          srcv.at[bj], semi.at[bj]).wait()
                    for gq in range(K // L):
                        dstv[bj, pl.ds(gq * L, L)] = (
                            dstv[bj, pl.ds(gq * L, L)] + goff)
                    pltpu.async_copy(x_hbm.at[dstv.at[bj]], rows.at[bj],
                                     semg.at[bj])

                @pl.when(jnp.logical_and(i >= 3, i < NCHUNK + 3))
                def _():
                    pltpu.make_async_copy(x_hbm.at[dstv.at[bk]], rows.at[bk],
                                          semg.at[bk]).wait()

                    def scale(gq, _):
                        e0 = gq * L
                        w16 = wv[bk, pl.ds(e0, L)]
                        for i2 in range(L):
                            we = w16[i2]
                            for jq in range(SEG):
                                seg = rows[bk, e0 + i2, pl.ds(jq * L, L)]
                                rows[bk, e0 + i2, pl.ds(jq * L, L)] = seg * we
                        return 0
                    lax.fori_loop(0, K // L, scale, 0)

                    pltpu.async_copy(rows.at[bk], acc_sh.at[srcv.at[bk]],
                                     sems.at[bk], add=True)

                    @pl.when(c == 0)
                    def _():
                        pltpu.async_copy(ones, cnt_sh.at[srcv.at[bk]],
                                         semc.at[bk], add=True)
            return 0
        lax.fori_loop(0, (NCHUNK + 3) // NBUF + 1, pipe, 0)

        plsc.subcore_barrier()
        pltpu.sync_copy(acc_sh.at[pl.ds(r0, RPT)], acc_out.at[c, pl.ds(r0, RPT)])

        @pl.when(c == 0)
        def _():
            pltpu.sync_copy(cnt_sh.at[pl.ds(r0, RPT)], cnt_out.at[pl.ds(r0, RPT)])

        @pl.when(s == NS - 1)
        def _():
            pltpu.sync_copy(acc_sh.at[pl.ds(RPT * NS, REM)],
                            acc_out.at[c, pl.ds(RPT * NS, REM)])

            @pl.when(c == 0)
            def _():
                pltpu.sync_copy(cnt_sh.at[pl.ds(RPT * NS, REM)],
                                cnt_out.at[pl.ds(RPT * NS, REM)])

    return k(xcols, src, dst, w)


def _tc_combine(x, acc, cnt, w1t, w2t, b2):
    N, D = x.shape
    OUT = w1t.shape[1]
    DH = D // 2
    BN = 400

    def body(x_ref, acc_ref, cnt_ref, w1_ref, w2_ref, b_ref, o_ref):
        xb = x_ref[...]
        a = jnp.concatenate([acc_ref[0], acc_ref[1]], axis=1)
        cn = cnt_ref[:, 0:1]
        agg = a * (1.0 / jnp.maximum(cn, 1.0))
        o_ref[...] = (
            jnp.dot(xb, w1_ref[...], preferred_element_type=jnp.float32)
            + jnp.dot(agg, w2_ref[...], preferred_element_type=jnp.float32)
            + b_ref[...]
        )

    return pl.pallas_call(
        body,
        grid=(N // BN,),
        in_specs=[
            pl.BlockSpec((BN, D), lambda i: (i, 0)),
            pl.BlockSpec((NC, BN, DH), lambda i: (0, i, 0)),
            pl.BlockSpec((BN, L), lambda i: (i, 0)),
            pl.BlockSpec((D, OUT), lambda i: (0, 0)),
            pl.BlockSpec((D, OUT), lambda i: (0, 0)),
            pl.BlockSpec((1, OUT), lambda i: (0, 0)),
        ],
        out_specs=pl.BlockSpec((BN, OUT), lambda i: (i, 0)),
        out_shape=jax.ShapeDtypeStruct((N, OUT), jnp.float32),
    )(x, acc, cnt, w1t, w2t, b2)


def kernel(x, edge_index, edge_weight, W, b):
    N, D = x.shape
    DH = D // 2
    src = edge_index[0]
    dst = edge_index[1]
    xcols = jnp.concatenate([x[:, :DH], x[:, DH:]], axis=0)
    acc, cnt = _sc_aggregate(xcols, src, dst, edge_weight, N)
    w1t = W[:, :D].T
    w2t = W[:, D:].T
    return _tc_combine(x, acc, cnt, w1t, w2t, b[None, :])


# speedup vs baseline: 9.9472x; 1.2062x over previous
import functools

import jax
import jax.numpy as jnp
from jax import lax
from jax.experimental import pallas as pl
from jax.experimental.pallas import tpu as pltpu
from jax.experimental.pallas import tpu_sc as plsc

NC = 2
NS = 16
L = 16


def _sc_aggregate(xcols, edge_index, w, n_nodes):
    N = n_nodes
    DH = xcols.shape[1]
    E = edge_index.shape[1]
    EPW = E // NS
    K = 80
    NBUF = 6
    NCHUNK = EPW // K
    assert EPW % K == 0 and E % NS == 0
    RPT = (N // NS) // 8 * 8
    REM = N - RPT * NS
    SEG = DH // L

    mesh = plsc.VectorSubcoreMesh(core_axis_name="c", subcore_axis_name="s")

    @functools.partial(
        pl.kernel,
        mesh=mesh,
        compiler_params=pltpu.CompilerParams(use_tc_tiling_on_sc=False),
        out_type=[
            jax.ShapeDtypeStruct((NC, N, DH), jnp.float32),
            jax.ShapeDtypeStruct((N, L), jnp.float32),
        ],
        scratch_types=[
            pltpu.VMEM((NBUF, K), jnp.int32),
            pltpu.VMEM((NBUF, K), jnp.int32),
            pltpu.VMEM((NBUF, K), jnp.float32),
            pltpu.VMEM((NBUF, K, 64), jnp.float32),
            pltpu.VMEM((K, L), jnp.float32),
            pltpu.VMEM((624, L), jnp.float32),
            pltpu.VMEM_SHARED((N, 64), jnp.float32),
            pltpu.VMEM_SHARED((N, L), jnp.float32),
            pltpu.SemaphoreType.DMA((NBUF,)),
            pltpu.SemaphoreType.DMA((NBUF,)),
            pltpu.SemaphoreType.DMA((NBUF,)),
            pltpu.SemaphoreType.DMA((NBUF,)),
        ],
    )
    def k(x_hbm, ei_hbm, w_hbm, acc_out, cnt_out,
          dstv, srcv, wv, rows, ones, zbuf, acc_sh, cnt_sh,
          semi, semg, sems, semc):
        c = lax.axis_index("c")
        s = lax.axis_index("s")

        zero = jnp.zeros((L,), jnp.float32)
        onehot = jnp.where(jnp.arange(L, dtype=jnp.int32) == 0, 1.0, 0.0)

        def init_rows(i, _):
            for j in range(SEG):
                rows[0, i, pl.ds(j * L, L)] = zero
            ones[i, :] = onehot
            return 0
        lax.fori_loop(0, K, init_rows, 0)

        def init_z(i, _):
            zbuf[i, :] = zero
            return 0
        lax.fori_loop(0, RPT, init_z, 0)

        r0 = pl.multiple_of(s * RPT, 8)
        done = 0
        while done < RPT:
            step = min(K, RPT - done)
            pltpu.sync_copy(rows.at[0, pl.ds(0, step)],
                            acc_sh.at[pl.ds(pl.multiple_of(r0 + done, 8), step)])
            done += step

        @pl.when(c == 0)
        def _():
            pltpu.sync_copy(zbuf.at[pl.ds(0, RPT)], cnt_sh.at[pl.ds(r0, RPT)])

        @pl.when(s == NS - 1)
        def _():
            pltpu.sync_copy(rows.at[0, pl.ds(0, REM)],
                            acc_sh.at[pl.ds(RPT * NS, REM)])

            @pl.when(c == 0)
            def _():
                pltpu.sync_copy(zbuf.at[pl.ds(0, REM)],
                                cnt_sh.at[pl.ds(RPT * NS, REM)])

        plsc.subcore_barrier()

        ebase = s * EPW
        goff = c * N

        def pipe(g, _):
            for b in range(NBUF):
                i = g * NBUF + b
                bj = (b - 2) % NBUF
                bk = (b - 3) % NBUF

                @pl.when(jnp.logical_and(i >= NBUF, i < NCHUNK + NBUF))
                def _():
                    pltpu.make_async_copy(
                        rows.at[b], acc_sh.at[srcv.at[b]], sems.at[b]).wait()

                    @pl.when(c == 0)
                    def _():
                        pltpu.make_async_copy(
                            ones, cnt_sh.at[srcv.at[b]], semc.at[b]).wait()

                @pl.when(i < NCHUNK)
                def _():
                    base = ebase + i * K
                    pltpu.async_copy(ei_hbm.at[1, pl.ds(base, K)], dstv.at[b],
                                     semi.at[b])
                    pltpu.async_copy(w_hbm.at[pl.ds(base, K)], wv.at[b],
                                     semi.at[b])
                    pltpu.async_copy(ei_hbm.at[0, pl.ds(base, K)], srcv.at[b],
                                     semi.at[b])

                @pl.when(jnp.logical_and(i >= 2, i < NCHUNK + 2))
                def _():
                    base = ebase + (i - 2) * K
                    pltpu.make_async_copy(ei_hbm.at[1, pl.ds(base, K)],
                                          dstv.at[bj], semi.at[bj]).wait()
                    pltpu.make_async_copy(w_hbm.at[pl.ds(base, K)],
                                          wv.at[bj], semi.at[bj]).wait()
                    pltpu.make_async_copy(ei_hbm.at[0, pl.ds(base, K)],
                                          srcv.at[bj], semi.at[bj]).wait()
                    for gq in range(K // L):
                        dstv[bj, pl.ds(gq * L, L)] = (
                            dstv[bj, pl.ds(gq * L, L)] + goff)
                    pltpu.async_copy(x_hbm.at[dstv.at[bj]], rows.at[bj],
                                     semg.at[bj])

                @pl.when(jnp.logical_and(i >= 3, i < NCHUNK + 3))
                def _():
                    pltpu.make_async_copy(x_hbm.at[dstv.at[bk]], rows.at[bk],
                                          semg.at[bk]).wait()

                    def scale(gq, _):
                        e0 = gq * L
                        w16 = wv[bk, pl.ds(e0, L)]
                        for i2 in range(L):
                            we = w16[i2]
                            for jq in range(SEG):
                                seg = rows[bk, e0 + i2, pl.ds(jq * L, L)]
                                rows[bk, e0 + i2, pl.ds(jq * L, L)] = seg * we
                        return 0
                    lax.fori_loop(0, K // L, scale, 0)

                    pltpu.async_copy(rows.at[bk], acc_sh.at[srcv.at[bk]],
                                     sems.at[bk], add=True)

                    @pl.when(c == 0)
                    def _():
                        pltpu.async_copy(ones, cnt_sh.at[srcv.at[bk]],
                                         semc.at[bk], add=True)
            return 0
        lax.fori_loop(0, -(-(NCHUNK + NBUF) // NBUF), pipe, 0)

        plsc.subcore_barrier()
        pltpu.sync_copy(acc_sh.at[pl.ds(r0, RPT)], acc_out.at[c, pl.ds(r0, RPT)])

        @pl.when(c == 0)
        def _():
            pltpu.sync_copy(cnt_sh.at[pl.ds(r0, RPT)], cnt_out.at[pl.ds(r0, RPT)])

        @pl.when(s == NS - 1)
        def _():
            pltpu.sync_copy(acc_sh.at[pl.ds(RPT * NS, REM)],
                            acc_out.at[c, pl.ds(RPT * NS, REM)])

            @pl.when(c == 0)
            def _():
                pltpu.sync_copy(cnt_sh.at[pl.ds(RPT * NS, REM)],
                                cnt_out.at[pl.ds(RPT * NS, REM)])

    return k(xcols, edge_index, w)


def _tc_combine(x, acc, cnt, W, b2):
    N, D = x.shape
    OUT = W.shape[0]
    DH = D // 2
    BN = 400

    def body(x_ref, acc_ref, cnt_ref, w_ref, b_ref, o_ref):
        xb = x_ref[...]
        a = jnp.concatenate([acc_ref[0], acc_ref[1]], axis=1)
        cn = cnt_ref[:, 0:1]
        agg = a * (1.0 / jnp.maximum(cn, 1.0))
        dn = (((1,), (1,)), ((), ()))
        o_ref[...] = (
            lax.dot_general(xb, w_ref[:, :D], dn,
                            preferred_element_type=jnp.float32)
            + lax.dot_general(agg, w_ref[:, D:], dn,
                              preferred_element_type=jnp.float32)
            + b_ref[...]
        )

    return pl.pallas_call(
        body,
        grid=(N // BN,),
        in_specs=[
            pl.BlockSpec((BN, D), lambda i: (i, 0)),
            pl.BlockSpec((NC, BN, DH), lambda i: (0, i, 0)),
            pl.BlockSpec((BN, L), lambda i: (i, 0)),
            pl.BlockSpec((OUT, 2 * D), lambda i: (0, 0)),
            pl.BlockSpec((1, OUT), lambda i: (0, 0)),
        ],
        out_specs=pl.BlockSpec((BN, OUT), lambda i: (i, 0)),
        out_shape=jax.ShapeDtypeStruct((N, OUT), jnp.float32),
    )(x, acc, cnt, W, b2)


def kernel(x, edge_index, edge_weight, W, b):
    N, D = x.shape
    DH = D // 2
    xcols = jnp.concatenate([x[:, :DH], x[:, DH:]], axis=0)
    acc, cnt = _sc_aggregate(xcols, edge_index, edge_weight, N)
    return _tc_combine(x, acc, cnt, W, b[None, :])


# speedup vs baseline: 11.0997x; 1.1159x over previous
import functools

import jax
import jax.numpy as jnp
from jax import lax
from jax.experimental import pallas as pl
from jax.experimental.pallas import tpu as pltpu
from jax.experimental.pallas import tpu_sc as plsc

NC = 2
NS = 16
L = 16


def _sc_aggregate(xcols, edge_index, w, n_nodes):
    N = n_nodes
    DH = xcols.shape[1]
    E = edge_index.shape[1]
    EPW = E // NS
    K = 80
    NBUF = 6
    NCHUNK = EPW // K
    assert EPW % K == 0 and E % NS == 0
    RPT = (N // NS) // 8 * 8
    REM = N - RPT * NS
    SEG = DH // L

    mesh = plsc.VectorSubcoreMesh(core_axis_name="c", subcore_axis_name="s")

    @functools.partial(
        pl.kernel,
        mesh=mesh,
        compiler_params=pltpu.CompilerParams(use_tc_tiling_on_sc=False),
        out_type=[
            jax.ShapeDtypeStruct((NC, N, DH), jnp.float32),
            jax.ShapeDtypeStruct((N, L), jnp.float32),
        ],
        scratch_types=[
            pltpu.VMEM((NBUF, 2, K), jnp.int32),
            pltpu.VMEM((NBUF, K), jnp.float32),
            pltpu.VMEM((NBUF, K, 64), jnp.float32),
            pltpu.VMEM((K, L), jnp.float32),
            pltpu.VMEM((624, L), jnp.float32),
            pltpu.VMEM_SHARED((N, 64), jnp.float32),
            pltpu.VMEM_SHARED((N, L), jnp.float32),
            pltpu.SemaphoreType.DMA((NBUF,)),
            pltpu.SemaphoreType.DMA((NBUF,)),
            pltpu.SemaphoreType.DMA((NBUF,)),
            pltpu.SemaphoreType.DMA((NBUF,)),
        ],
    )
    def k(x_hbm, ei_hbm, w_hbm, acc_out, cnt_out,
          idxv, wv, rows, ones, zbuf, acc_sh, cnt_sh,
          semi, semg, sems, semc):
        c = lax.axis_index("c")
        s = lax.axis_index("s")

        zero = jnp.zeros((L,), jnp.float32)
        onehot = jnp.where(jnp.arange(L, dtype=jnp.int32) == 0, 1.0, 0.0)

        def init_rows(i, _):
            for j in range(SEG):
                rows[0, i, pl.ds(j * L, L)] = zero
            ones[i, :] = onehot
            return 0
        lax.fori_loop(0, K, init_rows, 0)

        def init_z(i, _):
            zbuf[i, :] = zero
            return 0
        lax.fori_loop(0, RPT, init_z, 0)

        r0 = pl.multiple_of(s * RPT, 8)
        done = 0
        while done < RPT:
            step = min(K, RPT - done)
            pltpu.sync_copy(rows.at[0, pl.ds(0, step)],
                            acc_sh.at[pl.ds(pl.multiple_of(r0 + done, 8), step)])
            done += step

        @pl.when(c == 0)
        def _():
            pltpu.sync_copy(zbuf.at[pl.ds(0, RPT)], cnt_sh.at[pl.ds(r0, RPT)])

        @pl.when(s == NS - 1)
        def _():
            pltpu.sync_copy(rows.at[0, pl.ds(0, REM)],
                            acc_sh.at[pl.ds(RPT * NS, REM)])

            @pl.when(c == 0)
            def _():
                pltpu.sync_copy(zbuf.at[pl.ds(0, REM)],
                                cnt_sh.at[pl.ds(RPT * NS, REM)])

        plsc.subcore_barrier()

        ebase = s * EPW

        def pipe(g, _):
            for b in range(NBUF):
                i = g * NBUF + b
                bj = (b - 2) % NBUF
                bk = (b - 3) % NBUF

                @pl.when(jnp.logical_and(i >= NBUF, i < NCHUNK + NBUF))
                def _():
                    pltpu.make_async_copy(
                        rows.at[b], acc_sh.at[idxv.at[b, 0]], sems.at[b]).wait()

                    @pl.when(c == 0)
                    def _():
                        pltpu.make_async_copy(
                            ones, cnt_sh.at[idxv.at[b, 0]], semc.at[b]).wait()

                @pl.when(i < NCHUNK)
                def _():
                    base = ebase + i * K
                    pltpu.async_copy(ei_hbm.at[:, pl.ds(base, K)], idxv.at[b],
                                     semi.at[b])
                    pltpu.async_copy(w_hbm.at[pl.ds(base, K)], wv.at[b],
                                     semi.at[b])

                @pl.when(jnp.logical_and(i >= 2, i < NCHUNK + 2))
                def _():
                    base = ebase + (i - 2) * K
                    pltpu.make_async_copy(ei_hbm.at[:, pl.ds(base, K)],
                                          idxv.at[bj], semi.at[bj]).wait()
                    pltpu.make_async_copy(w_hbm.at[pl.ds(base, K)],
                                          wv.at[bj], semi.at[bj]).wait()
                    for gq in range(K // L):
                        dstv16 = idxv[bj, 1, pl.ds(gq * L, L)]
                        idxv[bj, 1, pl.ds(gq * L, L)] = dstv16 + dstv16 + c
                    pltpu.async_copy(x_hbm.at[idxv.at[bj, 1]], rows.at[bj],
                                     semg.at[bj])

                @pl.when(jnp.logical_and(i >= 3, i < NCHUNK + 3))
                def _():
                    pltpu.make_async_copy(x_hbm.at[idxv.at[bk, 1]],
                                          rows.at[bk], semg.at[bk]).wait()

                    def scale(gq, _):
                        e0 = gq * L
                        w16 = wv[bk, pl.ds(e0, L)]
                        for i2 in range(L):
                            we = w16[i2]
                            for jq in range(SEG):
                                seg = rows[bk, e0 + i2, pl.ds(jq * L, L)]
                                rows[bk, e0 + i2, pl.ds(jq * L, L)] = seg * we
                        return 0
                    lax.fori_loop(0, K // L, scale, 0)

                    pltpu.async_copy(rows.at[bk], acc_sh.at[idxv.at[bk, 0]],
                                     sems.at[bk], add=True)

                    @pl.when(c == 0)
                    def _():
                        pltpu.async_copy(ones, cnt_sh.at[idxv.at[bk, 0]],
                                         semc.at[bk], add=True)
            return 0
        lax.fori_loop(0, -(-(NCHUNK + NBUF) // NBUF), pipe, 0)

        plsc.subcore_barrier()
        pltpu.sync_copy(acc_sh.at[pl.ds(r0, RPT)], acc_out.at[c, pl.ds(r0, RPT)])

        @pl.when(c == 0)
        def _():
            pltpu.sync_copy(cnt_sh.at[pl.ds(r0, RPT)], cnt_out.at[pl.ds(r0, RPT)])

        @pl.when(s == NS - 1)
        def _():
            pltpu.sync_copy(acc_sh.at[pl.ds(RPT * NS, REM)],
                            acc_out.at[c, pl.ds(RPT * NS, REM)])

            @pl.when(c == 0)
            def _():
                pltpu.sync_copy(cnt_sh.at[pl.ds(RPT * NS, REM)],
                                cnt_out.at[pl.ds(RPT * NS, REM)])

    return k(xcols, edge_index, w)


def _tc_self(x, W, b2):
    N, D = x.shape
    OUT = W.shape[0]
    BN = 400

    def body(x_ref, w_ref, b_ref, o_ref):
        dn = (((1,), (1,)), ((), ()))
        o_ref[...] = lax.dot_general(
            x_ref[...], w_ref[:, :D], dn,
            preferred_element_type=jnp.float32) + b_ref[...]

    return pl.pallas_call(
        body,
        grid=(N // BN,),
        in_specs=[
            pl.BlockSpec((BN, D), lambda i: (i, 0)),
            pl.BlockSpec((OUT, 2 * D), lambda i: (0, 0)),
            pl.BlockSpec((1, OUT), lambda i: (0, 0)),
        ],
        out_specs=pl.BlockSpec((BN, OUT), lambda i: (i, 0)),
        out_shape=jax.ShapeDtypeStruct((N, OUT), jnp.float32),
    )(x, W, b2)


def _tc_combine(h, acc, cnt, W):
    N, D = h.shape
    OUT = W.shape[0]
    DH = acc.shape[2]
    BN = 400

    def body(h_ref, acc_ref, cnt_ref, w_ref, o_ref):
        a = jnp.concatenate([acc_ref[0], acc_ref[1]], axis=1)
        cn = cnt_ref[:, 0:1]
        agg = a * (1.0 / jnp.maximum(cn, 1.0))
        dn = (((1,), (1,)), ((), ()))
        o_ref[...] = h_ref[...] + lax.dot_general(
            agg, w_ref[:, 2 * DH:], dn, preferred_element_type=jnp.float32)

    return pl.pallas_call(
        body,
        grid=(N // BN,),
        in_specs=[
            pl.BlockSpec((BN, D), lambda i: (i, 0)),
            pl.BlockSpec((NC, BN, DH), lambda i: (0, i, 0)),
            pl.BlockSpec((BN, L), lambda i: (i, 0)),
            pl.BlockSpec((OUT, 4 * DH), lambda i: (0, 0)),
        ],
        out_specs=pl.BlockSpec((BN, OUT), lambda i: (i, 0)),
        out_shape=jax.ShapeDtypeStruct((N, OUT), jnp.float32),
    )(h, acc, cnt, W)


def kernel(x, edge_index, edge_weight, W, b):
    N, D = x.shape
    DH = D // 2
    xcols = x.reshape(2 * N, DH)
    acc, cnt = _sc_aggregate(xcols, edge_index, edge_weight, N)
    h = _tc_self(x, W, b[None, :])
    return _tc_combine(h, acc, cnt, W)
